# baseline (speedup 1.0000x reference)
import jax
import jax.numpy as jnp
from jax import lax
from jax.experimental import pallas as pl

B = 4096
HIDDEN = 2048
SPEC = 64
NLEAF = 8192
TOPK = 16
BM = 256
BM_PROJ = 1024
NSTACK = 5
NCH = NLEAF // 128

NEG_INF = float("-inf")


def _proj_body(x_ref, ws_ref, w3_ref, ps_ref, po_ref):
    ps_ref[...] = jnp.dot(x_ref[...], ws_ref[...],
                          preferred_element_type=jnp.float32)
    po_ref[...] = jnp.dot(x_ref[...], w3_ref[...],
                          preferred_element_type=jnp.float32)


def _score_block(sp_ref, po_ref, pd_ref, aux_ref, l2_ref, ls_ref, lp_ref):
    sim = jnp.dot(sp_ref[...], ls_ref[...], preferred_element_type=jnp.float32)
    og = jnp.dot(po_ref[...], lp_ref[...], preferred_element_type=jnp.float32)
    tb = jnp.dot(pd_ref[...], lp_ref[...], preferred_element_type=jnp.float32)
    od = aux_ref[:, 0:1]
    o2 = aux_ref[:, 1:2]
    t = tb - od
    d2 = l2_ref[0:1, :] - 2.0 * og + o2 - t * t
    return sim - d2


def _fast_body(sp_ref, po_ref, pd_ref, aux_ref, l2_ref, ls_ref, lp_ref,
               vals_ref, idx_ref, flag_ref):
    s = _score_block(sp_ref, po_ref, pd_ref, aux_ref, l2_ref, ls_ref, lp_ref)
    i32 = jnp.int32
    R = [jnp.full((BM, 128), NEG_INF, jnp.float32) for _ in range(NSTACK)]
    A = [jnp.zeros((BM, 128), i32) for _ in range(NSTACK)]
    for j in range(NCH):
        v = s[:, j * 128:(j + 1) * 128]
        gt = [v > R[i] for i in range(NSTACK)]
        newR = [jnp.where(gt[0], v, R[0])]
        newA = [jnp.where(gt[0], j, A[0])]
        for i in range(1, NSTACK):
            newR.append(jnp.where(gt[i - 1], R[i - 1],
                                  jnp.where(gt[i], v, R[i])))
            newA.append(jnp.where(gt[i - 1], A[i - 1],
                                  jnp.where(gt[i], j, A[i])))
        R, A = newR, newA

    liota = lax.broadcasted_iota(i32, (BM, 128), 1)
    pulls = jnp.zeros((BM, 128), i32)
    vals = []
    idxs = []
    for _ in range(TOPK):
        m = jnp.max(R[0], axis=1, keepdims=True)
        cand = jnp.where(R[0] == m, A[0] * 128 + liota, jnp.int32(1 << 30))
        g = jnp.min(cand, axis=1, keepdims=True)
        vals.append(m)
        idxs.append(g)
        lh = liota == (g & 127)
        for i in range(NSTACK - 1):
            R[i] = jnp.where(lh, R[i + 1], R[i])
            A[i] = jnp.where(lh, A[i + 1], A[i])
        R[NSTACK - 1] = jnp.where(lh, NEG_INF, R[NSTACK - 1])
        pulls = pulls + jnp.where(lh, 1, 0)
    vals_ref[...] = jnp.concatenate(vals, axis=1)
    idx_ref[...] = jnp.concatenate(idxs, axis=1)
    flag_ref[...] = jnp.full((8, 128), jnp.max(pulls), i32)


def _exact_body(sp_ref, po_ref, pd_ref, aux_ref, l2_ref, ls_ref, lp_ref,
                vals_ref, idx_ref):
    s = _score_block(sp_ref, po_ref, pd_ref, aux_ref, l2_ref, ls_ref, lp_ref)
    iota = lax.broadcasted_iota(jnp.int32, (BM, NLEAF), 1)
    vals = []
    idxs = []
    for _ in range(TOPK):
        m = jnp.max(s, axis=1, keepdims=True)
        cand = jnp.where(s == m, iota, NLEAF)
        ik = jnp.min(cand, axis=1, keepdims=True)
        vals.append(m)
        idxs.append(ik)
        s = jnp.where(iota == ik, NEG_INF, s)
    vals_ref[...] = jnp.concatenate(vals, axis=1)
    idx_ref[...] = jnp.concatenate(idxs, axis=1)


def kernel(x, W3, b3, Wspec, bspec, leaves_pos, leaves_spec):
    f32 = jnp.float32
    W3p = jnp.concatenate([W3.T, jnp.zeros((HIDDEN, 5), f32)], axis=1)
    Pspec, Porig = pl.pallas_call(
        _proj_body,
        grid=(B // BM_PROJ,),
        in_specs=[
            pl.BlockSpec((BM_PROJ, HIDDEN), lambda i: (i, 0)),
            pl.BlockSpec((HIDDEN, SPEC), lambda i: (0, 0)),
            pl.BlockSpec((HIDDEN, 8), lambda i: (0, 0)),
        ],
        out_specs=[
            pl.BlockSpec((BM_PROJ, SPEC), lambda i: (i, 0)),
            pl.BlockSpec((BM_PROJ, 8), lambda i: (i, 0)),
        ],
        out_shape=[
            jax.ShapeDtypeStruct((B, SPEC), f32),
            jax.ShapeDtypeStruct((B, 8), f32),
        ],
    )(x, Wspec.T, W3p)
    origins = Porig[:, :3] + b3
    dkey = jax.random.key(42)
    dirs = jax.random.normal(dkey, origins.shape, dtype=origins.dtype)
    dirs = dirs / jnp.linalg.norm(dirs, axis=-1, keepdims=True)
    spectral = Pspec + bspec

    od = jnp.sum(origins * dirs, axis=-1, keepdims=True)
    o2 = jnp.sum(origins ** 2, axis=-1, keepdims=True)
    aux = jnp.concatenate([od, o2, jnp.zeros((B, 6), f32)], axis=1)
    PO = jnp.concatenate([origins, jnp.zeros((B, 5), f32)], axis=1)
    PD = jnp.concatenate([dirs, jnp.zeros((B, 5), f32)], axis=1)
    l2 = jnp.sum(leaves_pos ** 2, axis=-1)[None, :]
    lsT = leaves_spec.T
    lpT = jnp.concatenate([leaves_pos.T, jnp.zeros((5, NLEAF), f32)], axis=0)

    in_specs = [
        pl.BlockSpec((BM, SPEC), lambda i: (i, 0)),
        pl.BlockSpec((BM, 8), lambda i: (i, 0)),
        pl.BlockSpec((BM, 8), lambda i: (i, 0)),
        pl.BlockSpec((BM, 8), lambda i: (i, 0)),
        pl.BlockSpec((1, NLEAF), lambda i: (0, 0)),
        pl.BlockSpec((SPEC, NLEAF), lambda i: (0, 0)),
        pl.BlockSpec((8, NLEAF), lambda i: (0, 0)),
    ]
    args = (spectral, PO, PD, aux, l2, lsT, lpT)

    vals, idx, flags = pl.pallas_call(
        _fast_body,
        grid=(B // BM,),
        in_specs=in_specs,
        out_specs=[
            pl.BlockSpec((BM, TOPK), lambda i: (i, 0)),
            pl.BlockSpec((BM, TOPK), lambda i: (i, 0)),
            pl.BlockSpec((8, 128), lambda i: (i, 0)),
        ],
        out_shape=[
            jax.ShapeDtypeStruct((B, TOPK), f32),
            jax.ShapeDtypeStruct((B, TOPK), jnp.int32),
            jax.ShapeDtypeStruct((B // BM * 8, 128), jnp.int32),
        ],
    )(*args)

    def _redo():
        return pl.pallas_call(
            _exact_body,
            grid=(B // BM,),
            in_specs=in_specs,
            out_specs=[
                pl.BlockSpec((BM, TOPK), lambda i: (i, 0)),
                pl.BlockSpec((BM, TOPK), lambda i: (i, 0)),
            ],
            out_shape=[
                jax.ShapeDtypeStruct((B, TOPK), f32),
                jax.ShapeDtypeStruct((B, TOPK), jnp.int32),
            ],
        )(*args)

    vals, idx = lax.cond(jnp.max(flags) >= NSTACK, _redo, lambda: (vals, idx))
    return vals, idx


# speedup vs baseline: 2.9266x; 2.9266x over previous
import jax
import jax.numpy as jnp
from jax import lax
from jax.experimental import pallas as pl

B = 4096
HIDDEN = 2048
SPEC = 64
NLEAF = 8192
TOPK = 16
BM = 256
BM_PROJ = 1024
NSTACK = 5
NCH = NLEAF // 128

NEG_INF = float("-inf")


def _proj_body(x_ref, ws_ref, w3_ref, ps_ref, po_ref):
    ps_ref[...] = jnp.dot(x_ref[...], ws_ref[...],
                          preferred_element_type=jnp.float32)
    po_ref[...] = jnp.dot(x_ref[...], w3_ref[...],
                          preferred_element_type=jnp.float32)


def _score_block(sp_ref, po_ref, pd_ref, aux_ref, l2_ref, ls_ref, lp_ref):
    sim = jnp.dot(sp_ref[...], ls_ref[...], preferred_element_type=jnp.float32)
    og = jnp.dot(po_ref[...], lp_ref[...], preferred_element_type=jnp.float32)
    tb = jnp.dot(pd_ref[...], lp_ref[...], preferred_element_type=jnp.float32)
    od = aux_ref[:, 0:1]
    o2 = aux_ref[:, 1:2]
    t = tb - od
    d2 = l2_ref[0:1, :] - 2.0 * og + o2 - t * t
    return sim - d2


def _extract_rolled(s):
    iota = lax.broadcasted_iota(jnp.int32, (BM, NLEAF), 1)
    kiota = lax.broadcasted_iota(jnp.int32, (BM, TOPK), 1)

    def body(k, carry):
        s, vals, idxs = carry
        m = jnp.max(s, axis=1, keepdims=True)
        cand = jnp.where(s == m, iota, NLEAF)
        ik = jnp.min(cand, axis=1, keepdims=True)
        vals = jnp.where(kiota == k, m, vals)
        idxs = jnp.where(kiota == k, ik, idxs)
        s = jnp.where(iota == ik, NEG_INF, s)
        return s, vals, idxs

    _, vals, idxs = lax.fori_loop(
        0, TOPK, body,
        (s, jnp.zeros((BM, TOPK), jnp.float32), jnp.zeros((BM, TOPK), jnp.int32)))
    return vals, idxs


def _fast_body(sp_ref, po_ref, pd_ref, aux_ref, l2_ref, ls_ref, lp_ref,
               vals_ref, idx_ref):
    s = _score_block(sp_ref, po_ref, pd_ref, aux_ref, l2_ref, ls_ref, lp_ref)
    i32 = jnp.int32
    R = [jnp.full((BM, 128), NEG_INF, jnp.float32) for _ in range(NSTACK)]
    A = [jnp.zeros((BM, 128), i32) for _ in range(NSTACK)]
    for j in range(NCH):
        v = s[:, j * 128:(j + 1) * 128]
        gt = [v > R[i] for i in range(NSTACK)]
        newR = [jnp.where(gt[0], v, R[0])]
        newA = [jnp.where(gt[0], j, A[0])]
        for i in range(1, NSTACK):
            newR.append(jnp.where(gt[i - 1], R[i - 1],
                                  jnp.where(gt[i], v, R[i])))
            newA.append(jnp.where(gt[i - 1], A[i - 1],
                                  jnp.where(gt[i], j, A[i])))
        R, A = newR, newA

    liota = lax.broadcasted_iota(i32, (BM, 128), 1)
    pulls = jnp.zeros((BM, 128), i32)
    vals = []
    idxs = []
    for _ in range(TOPK):
        m = jnp.max(R[0], axis=1, keepdims=True)
        cand = jnp.where(R[0] == m, A[0] * 128 + liota, jnp.int32(1 << 30))
        g = jnp.min(cand, axis=1, keepdims=True)
        vals.append(m)
        idxs.append(g)
        lh = liota == (g & 127)
        for i in range(NSTACK - 1):
            R[i] = jnp.where(lh, R[i + 1], R[i])
            A[i] = jnp.where(lh, A[i + 1], A[i])
        R[NSTACK - 1] = jnp.where(lh, NEG_INF, R[NSTACK - 1])
        pulls = pulls + jnp.where(lh, 1, 0)
    fast = (jnp.concatenate(vals, axis=1), jnp.concatenate(idxs, axis=1))

    exhausted = jnp.max(pulls) >= NSTACK
    out_vals, out_idx = lax.cond(exhausted,
                                 lambda: _extract_rolled(s),
                                 lambda: fast)
    vals_ref[...] = out_vals
    idx_ref[...] = out_idx


def kernel(x, W3, b3, Wspec, bspec, leaves_pos, leaves_spec):
    f32 = jnp.float32
    W3p = jnp.concatenate([W3.T, jnp.zeros((HIDDEN, 5), f32)], axis=1)
    Pspec, Porig = pl.pallas_call(
        _proj_body,
        grid=(B // BM_PROJ,),
        in_specs=[
            pl.BlockSpec((BM_PROJ, HIDDEN), lambda i: (i, 0)),
            pl.BlockSpec((HIDDEN, SPEC), lambda i: (0, 0)),
            pl.BlockSpec((HIDDEN, 8), lambda i: (0, 0)),
        ],
        out_specs=[
            pl.BlockSpec((BM_PROJ, SPEC), lambda i: (i, 0)),
            pl.BlockSpec((BM_PROJ, 8), lambda i: (i, 0)),
        ],
        out_shape=[
            jax.ShapeDtypeStruct((B, SPEC), f32),
            jax.ShapeDtypeStruct((B, 8), f32),
        ],
    )(x, Wspec.T, W3p)
    origins = Porig[:, :3] + b3
    dkey = jax.random.key(42)
    dirs = jax.random.normal(dkey, origins.shape, dtype=origins.dtype)
    dirs = dirs / jnp.linalg.norm(dirs, axis=-1, keepdims=True)
    spectral = Pspec + bspec

    od = jnp.sum(origins * dirs, axis=-1, keepdims=True)
    o2 = jnp.sum(origins ** 2, axis=-1, keepdims=True)
    aux = jnp.concatenate([od, o2, jnp.zeros((B, 6), f32)], axis=1)
    PO = jnp.concatenate([origins, jnp.zeros((B, 5), f32)], axis=1)
    PD = jnp.concatenate([dirs, jnp.zeros((B, 5), f32)], axis=1)
    l2 = jnp.sum(leaves_pos ** 2, axis=-1)[None, :]
    lsT = leaves_spec.T
    lpT = jnp.concatenate([leaves_pos.T, jnp.zeros((5, NLEAF), f32)], axis=0)

    in_specs = [
        pl.BlockSpec((BM, SPEC), lambda i: (i, 0)),
        pl.BlockSpec((BM, 8), lambda i: (i, 0)),
        pl.BlockSpec((BM, 8), lambda i: (i, 0)),
        pl.BlockSpec((BM, 8), lambda i: (i, 0)),
        pl.BlockSpec((1, NLEAF), lambda i: (0, 0)),
        pl.BlockSpec((SPEC, NLEAF), lambda i: (0, 0)),
        pl.BlockSpec((8, NLEAF), lambda i: (0, 0)),
    ]
    args = (spectral, PO, PD, aux, l2, lsT, lpT)

    vals, idx = pl.pallas_call(
        _fast_body,
        grid=(B // BM,),
        in_specs=in_specs,
        out_specs=[
            pl.BlockSpec((BM, TOPK), lambda i: (i, 0)),
            pl.BlockSpec((BM, TOPK), lambda i: (i, 0)),
        ],
        out_shape=[
            jax.ShapeDtypeStruct((B, TOPK), f32),
            jax.ShapeDtypeStruct((B, TOPK), jnp.int32),
        ],
    )(*args)
    return vals, idx
